# baseline (speedup 1.0000x reference)
import jax
import jax.numpy as jnp
from jax.experimental import pallas as pl
from jax.experimental.pallas import tpu as pltpu

T, N, D = 8, 50000, 128
ROWS = T * N
BLOCK = 16384
CHUNKS = BLOCK // D
SUB = 4
OBS_ROWS = ROWS // D


def _fused_kernel(obs_ref, x_ref, w0_ref, b0_ref, w1_ref,
                  mbig_ref, out_ref):
    obs_t = obs_ref[:].T
    mp_all = jnp.maximum(obs_t, 0.0).astype(jnp.bfloat16)
    mn_all = jnp.maximum(-obs_t, 0.0).astype(jnp.bfloat16)
    b0 = b0_ref[:]
    for c in range(0, CHUNKS, SUB):
        xc = x_ref[pl.ds(c * D, SUB * D), :]
        mp = jnp.concatenate(
            [mp_all[:, c + s:c + s + 1] for s in range(SUB)], axis=0)
        mn = jnp.concatenate(
            [mn_all[:, c + s:c + s + 1] for s in range(SUB)], axis=0)
        m3 = jnp.concatenate([mp, mn, jnp.ones_like(mp)], axis=1)
        aux = jnp.dot(m3, mbig_ref[:], preferred_element_type=jnp.float32)
        h = jnp.dot(xc.astype(jnp.bfloat16), w0_ref[:],
                    preferred_element_type=jnp.float32) + b0
        h = jnp.maximum(h, 0.0)
        hm = (h * aux[:, :2 * D]).astype(jnp.bfloat16)
        o = jnp.dot(hm, w1_ref[:], preferred_element_type=jnp.float32)
        o = o + aux[:, 2 * D:3 * D] + aux[:, 3 * D:] * xc
        out_ref[pl.ds(c * D, SUB * D), :] = o


def kernel(node_data, observations, pos_W0, pos_b0, pos_W1, pos_b1,
           neg_W0, neg_b0, neg_W1, neg_b1):
    x = node_data.reshape(ROWS, D)
    obs = observations.reshape(OBS_ROWS, D).astype(jnp.float32)
    w0 = jnp.concatenate([pos_W0, neg_W0], axis=1).astype(jnp.bfloat16)
    b0 = jnp.concatenate([pos_b0, neg_b0]).reshape(1, 2 * D)
    w1 = jnp.concatenate([pos_W1, neg_W1], axis=0).astype(jnp.bfloat16)
    ones = jnp.ones((1, D), jnp.float32)
    zeros = jnp.zeros((1, D), jnp.float32)
    mbig = jnp.concatenate([
        jnp.concatenate([ones, zeros, pos_b1.reshape(1, D), -ones], axis=1),
        jnp.concatenate([zeros, ones, neg_b1.reshape(1, D), -ones], axis=1),
        jnp.concatenate([zeros, zeros, zeros, ones], axis=1),
    ], axis=0).astype(jnp.bfloat16)

    grid = pl.cdiv(ROWS, BLOCK)
    out = pl.pallas_call(
        _fused_kernel,
        grid=(grid,),
        in_specs=[
            pl.BlockSpec((CHUNKS, D), lambda i: (i, 0)),
            pl.BlockSpec((BLOCK, D), lambda i: (i, 0)),
            pl.BlockSpec((D, 2 * D), lambda i: (0, 0)),
            pl.BlockSpec((1, 2 * D), lambda i: (0, 0)),
            pl.BlockSpec((2 * D, D), lambda i: (0, 0)),
            pl.BlockSpec((3, 4 * D), lambda i: (0, 0)),
        ],
        out_specs=pl.BlockSpec((BLOCK, D), lambda i: (i, 0)),
        out_shape=jax.ShapeDtypeStruct((ROWS, D), jnp.float32),
        compiler_params=pltpu.CompilerParams(
            dimension_semantics=("parallel",),
        ),
    )(obs, x, w0, b0, w1, mbig)
    return out.reshape(T, N, D)


# speedup vs baseline: 1.2041x; 1.2041x over previous
import jax
import jax.numpy as jnp
from jax.experimental import pallas as pl
from jax.experimental.pallas import tpu as pltpu

T, N, D = 8, 50000, 128
ROWS = T * N
BLOCK = 16384
CHUNKS = BLOCK // D
SUB = 4
OBS_ROWS = ROWS // D


def _fused_kernel(obs_ref, x_ref, w0_ref, b0_ref, w1_ref,
                  b1p_ref, b1n_ref, out_ref):
    obs_t = obs_ref[:].T
    mp_all = jnp.maximum(obs_t, 0.0)
    mn_all = jnp.maximum(-obs_t, 0.0)
    b0 = b0_ref[:]
    b1p = b1p_ref[:]
    b1n = b1n_ref[:]
    for c in range(0, CHUNKS, SUB):
        xc = x_ref[pl.ds(c * D, SUB * D), :]
        mp = jnp.concatenate(
            [mp_all[:, c + s:c + s + 1] for s in range(SUB)], axis=0)
        mn = jnp.concatenate(
            [mn_all[:, c + s:c + s + 1] for s in range(SUB)], axis=0)
        h = jnp.dot(xc.astype(jnp.bfloat16), w0_ref[:],
                    preferred_element_type=jnp.float32) + b0
        h = jnp.maximum(h, 0.0)
        hm = jnp.concatenate(
            [(h[:, :D] * mp).astype(jnp.bfloat16),
             (h[:, D:] * mn).astype(jnp.bfloat16)], axis=1)
        o = jnp.dot(hm, w1_ref[:], preferred_element_type=jnp.float32)
        o = o + mp * b1p + mn * b1n + (1.0 - mp - mn) * xc
        out_ref[pl.ds(c * D, SUB * D), :] = o


def kernel(node_data, observations, pos_W0, pos_b0, pos_W1, pos_b1,
           neg_W0, neg_b0, neg_W1, neg_b1):
    x = node_data.reshape(ROWS, D)
    obs = observations.reshape(OBS_ROWS, D).astype(jnp.float32)
    w0 = jnp.concatenate([pos_W0, neg_W0], axis=1).astype(jnp.bfloat16)
    b0 = jnp.concatenate([pos_b0, neg_b0]).reshape(1, 2 * D)
    w1 = jnp.concatenate([pos_W1, neg_W1], axis=0).astype(jnp.bfloat16)
    b1p = pos_b1.reshape(1, D)
    b1n = neg_b1.reshape(1, D)

    grid = pl.cdiv(ROWS, BLOCK)
    out = pl.pallas_call(
        _fused_kernel,
        grid=(grid,),
        in_specs=[
            pl.BlockSpec((CHUNKS, D), lambda i: (i, 0)),
            pl.BlockSpec((BLOCK, D), lambda i: (i, 0)),
            pl.BlockSpec((D, 2 * D), lambda i: (0, 0)),
            pl.BlockSpec((1, 2 * D), lambda i: (0, 0)),
            pl.BlockSpec((2 * D, D), lambda i: (0, 0)),
            pl.BlockSpec((1, D), lambda i: (0, 0)),
            pl.BlockSpec((1, D), lambda i: (0, 0)),
        ],
        out_specs=pl.BlockSpec((BLOCK, D), lambda i: (i, 0)),
        out_shape=jax.ShapeDtypeStruct((ROWS, D), jnp.float32),
        compiler_params=pltpu.CompilerParams(
            dimension_semantics=("parallel",),
        ),
    )(obs, x, w0, b0, w1, b1p, b1n)
    return out.reshape(T, N, D)
